# Initial kernel scaffold; baseline (speedup 1.0000x reference)
import jax
import jax.numpy as jnp
from jax.experimental import pallas as pl


def kernel(x, Wa, ba, Wb, bb, Wc, bc):
    raise NotImplementedError("write your pallas kernel here")


# speedup vs baseline: 9.2750x; 9.2750x over previous
import functools

import jax
import jax.numpy as jnp
from jax import lax
from jax.experimental import pallas as pl

B, N, L, K = 4, 2048, 256, 16
ROWS = 256


def _scores_body(x_ref, wa_ref, ba_ref, wb_ref, bb_ref, wc_ref, bc_ref, out_ref):
    xb = x_ref[0]
    a = jnp.tanh(jnp.dot(xb, wa_ref[...], preferred_element_type=jnp.float32)
                 + ba_ref[...][None, :])
    b = jax.nn.sigmoid(jnp.dot(xb, wb_ref[...], preferred_element_type=jnp.float32)
                       + bb_ref[...][None, :])
    logits = (jnp.dot(a * b, wc_ref[...], preferred_element_type=jnp.float32)
              + bc_ref[...][None, :])
    m = jnp.max(logits, axis=0, keepdims=True)
    e = jnp.exp(logits - m)
    s = jnp.sum(e, axis=0, keepdims=True)
    out_ref[...] = (e / s)[None]


def _topk_body(acol_ref, arow_ref, out_ref):
    ac = acol_ref[0]
    aa = arow_ref[0]
    diff = ac - aa
    d = jnp.sqrt(diff * diff)
    iota_j = lax.broadcasted_iota(jnp.int32, (ROWS, N), 1)
    picks = []
    for _ in range(K):
        m = jnp.min(d, axis=1, keepdims=True)
        eq = d == m
        cand = jnp.where(eq, iota_j, N)
        amin = jnp.min(cand, axis=1, keepdims=True)
        picks.append(amin)
        d = jnp.where(iota_j == amin, jnp.inf, d)
    out_ref[0] = jnp.concatenate(picks, axis=1)


@jax.jit
def kernel(x, Wa, ba, Wb, bb, Wc, bc):
    scores = pl.pallas_call(
        _scores_body,
        grid=(B,),
        in_specs=[
            pl.BlockSpec((1, N, L), lambda b: (b, 0, 0)),
            pl.BlockSpec((L, L), lambda b: (0, 0)),
            pl.BlockSpec((L,), lambda b: (0,)),
            pl.BlockSpec((L, L), lambda b: (0, 0)),
            pl.BlockSpec((L,), lambda b: (0,)),
            pl.BlockSpec((L, 1), lambda b: (0, 0)),
            pl.BlockSpec((1,), lambda b: (0,)),
        ],
        out_specs=pl.BlockSpec((1, N, 1), lambda b: (b, 0, 0)),
        out_shape=jax.ShapeDtypeStruct((B, N, 1), jnp.float32),
    )(x, Wa, ba, Wb, bb, Wc, bc)

    a_col = scores
    a_row = scores.reshape(B, 1, N)

    nn_idx = pl.pallas_call(
        _topk_body,
        grid=(B, N // ROWS),
        in_specs=[
            pl.BlockSpec((1, ROWS, 1), lambda b, t: (b, t, 0)),
            pl.BlockSpec((1, 1, N), lambda b, t: (b, 0, 0)),
        ],
        out_specs=pl.BlockSpec((1, ROWS, K), lambda b, t: (b, t, 0)),
        out_shape=jax.ShapeDtypeStruct((B, N, K), jnp.int32),
    )(a_col, a_row)

    center_idx = jnp.broadcast_to(
        jnp.arange(N, dtype=nn_idx.dtype)[None, :, None], (B, N, K))
    return jnp.stack((nn_idx, center_idx), axis=0)


# speedup vs baseline: 39.2691x; 4.2339x over previous
import functools

import jax
import jax.numpy as jnp
from jax import lax
from jax.experimental import pallas as pl
from jax.experimental.pallas import tpu as pltpu
from jax.experimental.pallas import tpu_sc as plsc

B, N, L, K = 4, 2048, 256, 16
ROWS = 256

NC, NS, LN = 2, 16, 16
NW = NC * NS
WPB = NW // B
RPW = N // WPB
GRP = RPW // LN


def _scores_body(x_ref, wa_ref, ba_ref, wb_ref, bb_ref, wc_ref, bc_ref, out_ref):
    xb = x_ref[0]
    a = jnp.tanh(jnp.dot(xb, wa_ref[...], preferred_element_type=jnp.float32)
                 + ba_ref[...][None, :])
    b = jax.nn.sigmoid(jnp.dot(xb, wb_ref[...], preferred_element_type=jnp.float32)
                       + bb_ref[...][None, :])
    logits = (jnp.dot(a * b, wc_ref[...], preferred_element_type=jnp.float32)
              + bc_ref[...][None, :])
    m = jnp.max(logits, axis=0, keepdims=True)
    e = jnp.exp(logits - m)
    s = jnp.sum(e, axis=0, keepdims=True)
    out_ref[...] = (e / s)[None]


def _ranks_body(acol_ref, arow_ref, out_ref):
    ac = acol_ref[0]
    aa = arow_ref[0]
    t = pl.program_id(1)
    jj = lax.broadcasted_iota(jnp.int32, (ROWS, N), 1)
    ii = lax.broadcasted_iota(jnp.int32, (ROWS, N), 0) + t * ROWS
    lt = (aa < ac).astype(jnp.int32)
    eq_lt = ((aa == ac) & (jj < ii)).astype(jnp.int32)
    rank = jnp.sum(lt + eq_lt, axis=1)
    out_ref[0] = rank[:, None]


def _sc_knn_body(vals_hbm, ranks_hbm, out_hbm,
                 vals_v, ranks_v, sval_v, sidx_v, otile_v):
    cid = lax.axis_index("c")
    sid = lax.axis_index("s")
    wid = sid * NC + cid
    b = lax.rem(wid, B)
    seg = lax.div(wid, B)

    pltpu.sync_copy(vals_hbm.at[pl.ds(b * N, N)], vals_v)
    pltpu.sync_copy(ranks_hbm.at[pl.ds(b * N, N)], ranks_v)

    def scat_body(i, carry):
        v = vals_v[pl.ds(i * LN, LN)]
        r = ranks_v[pl.ds(i * LN, LN)]
        idx = lax.iota(jnp.int32, LN) + i * LN
        plsc.store_scatter(sval_v, [r], v)
        plsc.store_scatter(sidx_v, [r], idx)
        return carry

    lax.fori_loop(0, N // LN, scat_body, 0)

    lane = lax.iota(jnp.int32, LN)
    inf = jnp.full((LN,), jnp.inf, jnp.float32)

    def grp_body(g, carry):
        base = seg * RPW + g * LN
        vi = vals_v[pl.ds(base, LN)]
        my_rank = ranks_v[pl.ds(base, LN)]
        l = my_rank
        h = my_rank + 1
        for t in range(K):
            lvalid = l >= 0
            hvalid = h < N
            lc = jnp.maximum(l, 0)
            hc = jnp.minimum(h, N - 1)
            vl = plsc.load_gather(sval_v, [lc])
            vh = plsc.load_gather(sval_v, [hc])
            il = plsc.load_gather(sidx_v, [lc])
            ih = plsc.load_gather(sidx_v, [hc])
            dl = jnp.where(lvalid, jnp.abs(vi - vl), inf)
            dh = jnp.where(hvalid, jnp.abs(vi - vh), inf)
            pick_l = (dl < dh) | ((dl == dh) & (il < ih))
            picked = jnp.where(pick_l, il, ih)
            plsc.store_scatter(otile_v, [lane * K + t], picked)
            l = jnp.where(pick_l, l - 1, l)
            h = jnp.where(pick_l, h, h + 1)
        pltpu.sync_copy(otile_v, out_hbm.at[pl.ds((b * N + base) * K, LN * K)])
        return carry

    lax.fori_loop(0, GRP, grp_body, 0)


@jax.jit
def kernel(x, Wa, ba, Wb, bb, Wc, bc):
    scores = pl.pallas_call(
        _scores_body,
        grid=(B,),
        in_specs=[
            pl.BlockSpec((1, N, L), lambda b: (b, 0, 0)),
            pl.BlockSpec((L, L), lambda b: (0, 0)),
            pl.BlockSpec((L,), lambda b: (0,)),
            pl.BlockSpec((L, L), lambda b: (0, 0)),
            pl.BlockSpec((L,), lambda b: (0,)),
            pl.BlockSpec((L, 1), lambda b: (0, 0)),
            pl.BlockSpec((1,), lambda b: (0,)),
        ],
        out_specs=pl.BlockSpec((1, N, 1), lambda b: (b, 0, 0)),
        out_shape=jax.ShapeDtypeStruct((B, N, 1), jnp.float32),
    )(x, Wa, ba, Wb, bb, Wc, bc)

    a_col = scores
    a_row = scores.reshape(B, 1, N)

    ranks = pl.pallas_call(
        _ranks_body,
        grid=(B, N // ROWS),
        in_specs=[
            pl.BlockSpec((1, ROWS, 1), lambda b, t: (b, t, 0)),
            pl.BlockSpec((1, 1, N), lambda b, t: (b, 0, 0)),
        ],
        out_specs=pl.BlockSpec((1, ROWS, 1), lambda b, t: (b, t, 0)),
        out_shape=jax.ShapeDtypeStruct((B, N, 1), jnp.int32),
    )(a_col, a_row)

    sc_knn = pl.kernel(
        _sc_knn_body,
        out_type=jax.ShapeDtypeStruct((B * N * K,), jnp.int32),
        mesh=plsc.VectorSubcoreMesh(core_axis_name="c", subcore_axis_name="s"),
        compiler_params=pltpu.CompilerParams(needs_layout_passes=False),
        scratch_types=[
            pltpu.VMEM((N,), jnp.float32),
            pltpu.VMEM((N,), jnp.int32),
            pltpu.VMEM((N,), jnp.float32),
            pltpu.VMEM((N,), jnp.int32),
            pltpu.VMEM((LN * K,), jnp.int32),
        ],
    )
    nn_flat = sc_knn(scores.reshape(B * N), ranks.reshape(B * N))

    nn_idx = nn_flat.reshape(B, N, K)
    center_idx = jnp.broadcast_to(
        jnp.arange(N, dtype=nn_idx.dtype)[None, :, None], (B, N, K))
    return jnp.stack((nn_idx, center_idx), axis=0)
